# baseline (speedup 1.0000x reference)
import functools

import jax
import jax.numpy as jnp
from jax.experimental import pallas as pl
from jax.experimental.pallas import tpu as pltpu

_SLOPE = 0.01
_BINS = (1, 2, 4, 8, 16)
_VMEM = 64 * 1024 * 1024


def _lrelu(v):
    return jnp.where(v >= 0, v, _SLOPE * v)


def _gconv(x, w, *, cin, nout, left_slot, pool, out_dtype=jnp.bfloat16):
    F, h, G, L = x.shape
    cout = w.shape[1] // nout
    xp = jnp.pad(x, ((0, 0), (1, 1), (1, 1), (0, 0)))
    pieces = []
    for kh in range(3):
        row = xp[:, kh:kh + h]
        pieces += [row[:, :, 0:G, left_slot * cin:(left_slot + 1) * cin],
                   row[:, :, 1:G + 1, :],
                   row[:, :, 2:G + 2, 0:cin]]
    patch = jnp.concatenate(pieces, axis=-1)
    y = jnp.dot(patch.reshape(F * h * G, patch.shape[-1]), w,
                preferred_element_type=jnp.float32)
    y = _lrelu(y).astype(out_dtype).reshape(F, h, G, nout * cout)
    if pool:
        y = y.reshape(F, h // 2, 2, G, nout * cout)
        y = jnp.maximum(y[:, :, 0], y[:, :, 1])
        y = jnp.maximum(y, jnp.roll(y, -cout, axis=-1))
    return y


def _stack_kernel(x_ref, w1_ref, w2_ref, w3_ref, w4_ref, w5_ref, w6_ref,
                  o_ref, *, fb, H, W):
    G = W // 8
    xg = x_ref[0, 0].reshape(fb, H + 4, G, 8).astype(jnp.bfloat16)
    xp = jnp.pad(xg, ((0, 0), (0, 0), (1, 1), (0, 0)))
    pieces = []
    for kh in range(5):
        row = xp[:, kh:kh + H]
        pieces += [row[:, :, 0:G, 6:8],
                   row[:, :, 1:G + 1, :],
                   row[:, :, 2:G + 2, 0:2]]
    patch = jnp.concatenate(pieces, axis=-1)
    y = jnp.dot(patch.reshape(fb * H * G, 60), w1_ref[...],
                preferred_element_type=jnp.float32)
    y = _lrelu(y).astype(jnp.bfloat16).reshape(fb, H, G, 8 * 32)

    y = _gconv(y, w2_ref[...], cin=32, nout=8, left_slot=7, pool=True)
    y = _gconv(y, w3_ref[...], cin=32, nout=4, left_slot=6, pool=False)
    y = _gconv(y, w4_ref[...], cin=64, nout=4, left_slot=3, pool=True)
    y = _gconv(y, w5_ref[...], cin=64, nout=2, left_slot=2, pool=False)
    y = _gconv(y, w6_ref[...], cin=128, nout=2, left_slot=1, pool=False,
               out_dtype=jnp.float32)

    m = jnp.max(y, axis=0)
    j = pl.program_id(1)

    @pl.when(j == 0)
    def _():
        o_ref[0] = m

    @pl.when(j > 0)
    def _():
        o_ref[0] = jnp.maximum(o_ref[0], m)


def _conv_stack(x, ws, *, fb):
    n, s, H, W = x.shape
    G = W // 8
    xp = jnp.pad(x, ((0, 0), (0, 0), (2, 2), (0, 0)))
    xp = xp.reshape(n, s // fb, fb, H + 4, W)
    w_specs = [pl.BlockSpec(w.shape, lambda i, j, nd=w.ndim: (0,) * nd)
               for w in ws]
    kern = functools.partial(_stack_kernel, fb=fb, H=H, W=W)
    return pl.pallas_call(
        kern,
        out_shape=jax.ShapeDtypeStruct((n, H // 4, G, 256), jnp.float32),
        grid=(n, s // fb),
        in_specs=[pl.BlockSpec((1, 1, fb, H + 4, W),
                               lambda i, j: (i, j, 0, 0, 0))] + w_specs,
        out_specs=pl.BlockSpec((1, H // 4, G, 256), lambda i, j: (i, 0, 0, 0)),
        compiler_params=pltpu.CompilerParams(
            dimension_semantics=("parallel", "arbitrary"),
            vmem_limit_bytes=_VMEM),
    )(xp, *ws)


def _head_kernel(g_ref, w_ref, b_ref, o_ref, acc_ref):
    k = pl.program_id(0)

    @pl.when(k == 0)
    def _():
        acc_ref[...] = jnp.zeros_like(acc_ref)

    acc_ref[...] += jnp.dot(g_ref[...].astype(jnp.bfloat16),
                            w_ref[...].astype(jnp.bfloat16),
                            preferred_element_type=jnp.float32)

    @pl.when(k == pl.num_programs(0) - 1)
    def _():
        z = acc_ref[...] + b_ref[...]
        dout = z.shape[1]
        off = 0
        for nb in _BINS:
            L = dout // nb
            ssum = z[:, :L]
            smax = z[:, :L]
            for b in range(1, nb):
                seg = z[:, b * L:(b + 1) * L]
                ssum = ssum + seg
                smax = jnp.maximum(smax, seg)
            o_ref[:, off:off + L] = ssum * (1.0 / nb) + smax
            off += L


def _head(g_flat, fc_w, fc_b):
    n, din = g_flat.shape
    dout = fc_w.shape[1]
    feat = sum(dout // b for b in _BINS)
    dk = max(d for d in (4096, 2048, 1024, 512, 256, 128, din)
             if din % d == 0 and d <= din)
    return pl.pallas_call(
        _head_kernel,
        out_shape=jax.ShapeDtypeStruct((n, feat), jnp.float32),
        grid=(din // dk,),
        in_specs=[pl.BlockSpec((n, dk), lambda k: (0, k)),
                  pl.BlockSpec((dk, dout), lambda k: (k, 0)),
                  pl.BlockSpec((1, dout), lambda k: (0, 0))],
        out_specs=pl.BlockSpec((n, feat), lambda k: (0, 0)),
        scratch_shapes=[pltpu.VMEM((n, dout), jnp.float32)],
        compiler_params=pltpu.CompilerParams(
            dimension_semantics=("arbitrary",),
            vmem_limit_bytes=_VMEM),
    )(g_flat, fc_w, fc_b.reshape(1, dout))


def _gw(w, gout, S, sigma):
    _, _, cin, cout = w.shape
    out = jnp.zeros((3, S + 2, cin, gout, cout), w.dtype)
    for j in range(gout):
        for k in range(3):
            d = sigma * (j - 1 + k)
            ds = 0 if d < 0 else (S + 1 if d >= S else 1 + d)
            out = out.at[:, ds, :, j, :].set(w[:, k, :, :])
    return out.reshape(3 * (S + 2) * cin, gout * cout).astype(jnp.bfloat16)


def _gw1(w, g):
    _, _, _, cout = w.shape
    out = jnp.zeros((5, g + 4, g, cout), w.dtype)
    for p in range(g):
        for k in range(5):
            out = out.at[:, p + k, p, :].set(w[:, k, 0, :])
    return out.reshape(5 * (g + 4), g * cout).astype(jnp.bfloat16)


def kernel(l1_w, l2_w, l3_w, l4_w, l5_w, l6_w, fc_w, fc_b, x):
    n, s, H, W = x.shape
    fb = max(d for d in range(1, 11) if s % d == 0)
    ws = [
        _gw1(l1_w, 8),
        _gw(l2_w, 8, 8, 1),
        _gw(l3_w, 4, 8, 2),
        _gw(l4_w, 4, 4, 1),
        _gw(l5_w, 2, 4, 2),
        _gw(l6_w, 2, 2, 1),
    ]
    g = _conv_stack(x, ws, fb=fb)
    feat = _head(g.reshape(n, -1), fc_w, fc_b)
    return feat[:, None, :], None


# speedup vs baseline: 1.0726x; 1.0726x over previous
import functools

import jax
import jax.numpy as jnp
from jax.experimental import pallas as pl
from jax.experimental.pallas import tpu as pltpu

_SLOPE = 0.01
_BINS = (1, 2, 4, 8, 16)
_VMEM = 64 * 1024 * 1024


def _lrelu(v):
    return jnp.where(v >= 0, v, _SLOPE * v)


def _gconv(x, w, *, cin, nout, left_slot, pool, out_dtype=jnp.bfloat16):
    F, h, G, L = x.shape
    cout = w.shape[1] // nout
    xp = jnp.pad(x, ((0, 0), (1, 1), (1, 1), (0, 0)))
    rows = [xp[:, kh:kh + h] for kh in range(3)]
    pieces = [r[:, :, 1:G + 1, :] for r in rows]
    for r in rows:
        pieces += [r[:, :, 0:G, left_slot * cin:(left_slot + 1) * cin],
                   r[:, :, 2:G + 2, 0:cin]]
    patch = jnp.concatenate(pieces, axis=-1)
    y = jnp.dot(patch.reshape(F * h * G, patch.shape[-1]), w,
                preferred_element_type=jnp.float32)
    y = _lrelu(y).astype(out_dtype).reshape(F, h, G, nout * cout)
    if pool:
        y = y.reshape(F, h // 2, 2, G, nout * cout)
        y = jnp.maximum(y[:, :, 0], y[:, :, 1])
        y = jnp.maximum(y, jnp.roll(y, -cout, axis=-1))
    return y


def _stack_kernel(x_ref, w1_ref, w2_ref, w3_ref, w4_ref, w5_ref, w6_ref,
                  o_ref, *, fb, H, W):
    G = W // 8
    xg = x_ref[0, 0].reshape(fb, H + 4, G, 8).astype(jnp.bfloat16)
    xp = jnp.pad(xg, ((0, 0), (0, 0), (1, 1), (0, 0)))
    pieces = []
    for kh in range(5):
        row = xp[:, kh:kh + H]
        pieces += [row[:, :, 0:G, 6:8],
                   row[:, :, 1:G + 1, :],
                   row[:, :, 2:G + 2, 0:2]]
    patch = jnp.concatenate(pieces, axis=-1)
    y = jnp.dot(patch.reshape(fb * H * G, 60), w1_ref[...],
                preferred_element_type=jnp.float32)
    y = _lrelu(y).astype(jnp.bfloat16).reshape(fb, H, G, 8 * 32)

    y = _gconv(y, w2_ref[...], cin=32, nout=8, left_slot=7, pool=True)
    y = _gconv(y, w3_ref[...], cin=32, nout=4, left_slot=6, pool=False)
    y = _gconv(y, w4_ref[...], cin=64, nout=4, left_slot=3, pool=True)
    y = _gconv(y, w5_ref[...], cin=64, nout=2, left_slot=2, pool=False)
    y = _gconv(y, w6_ref[...], cin=128, nout=2, left_slot=1, pool=False,
               out_dtype=jnp.float32)

    m = jnp.max(y, axis=0)
    j = pl.program_id(1)

    @pl.when(j == 0)
    def _():
        o_ref[0] = m

    @pl.when(j > 0)
    def _():
        o_ref[0] = jnp.maximum(o_ref[0], m)


def _conv_stack(x, ws, *, fb):
    n, s, H, W = x.shape
    G = W // 8
    xp = jnp.pad(x, ((0, 0), (0, 0), (2, 2), (0, 0)))
    xp = xp.reshape(n, s // fb, fb, H + 4, W)
    w_specs = [pl.BlockSpec(w.shape, lambda i, j, nd=w.ndim: (0,) * nd)
               for w in ws]
    kern = functools.partial(_stack_kernel, fb=fb, H=H, W=W)
    return pl.pallas_call(
        kern,
        out_shape=jax.ShapeDtypeStruct((n, H // 4, G, 256), jnp.float32),
        grid=(n, s // fb),
        in_specs=[pl.BlockSpec((1, 1, fb, H + 4, W),
                               lambda i, j: (i, j, 0, 0, 0))] + w_specs,
        out_specs=pl.BlockSpec((1, H // 4, G, 256), lambda i, j: (i, 0, 0, 0)),
        compiler_params=pltpu.CompilerParams(
            dimension_semantics=("parallel", "arbitrary"),
            vmem_limit_bytes=_VMEM),
    )(xp, *ws)


def _head_kernel(g_ref, w_ref, b_ref, o_ref, acc_ref):
    k = pl.program_id(0)

    @pl.when(k == 0)
    def _():
        acc_ref[...] = jnp.zeros_like(acc_ref)

    acc_ref[...] += jnp.dot(g_ref[...].astype(jnp.bfloat16),
                            w_ref[...].astype(jnp.bfloat16),
                            preferred_element_type=jnp.float32)

    @pl.when(k == pl.num_programs(0) - 1)
    def _():
        z = acc_ref[...] + b_ref[...]
        dout = z.shape[1]
        off = 0
        for nb in _BINS:
            L = dout // nb
            ssum = z[:, :L]
            smax = z[:, :L]
            for b in range(1, nb):
                seg = z[:, b * L:(b + 1) * L]
                ssum = ssum + seg
                smax = jnp.maximum(smax, seg)
            o_ref[:, off:off + L] = ssum * (1.0 / nb) + smax
            off += L


def _head(g_flat, fc_w, fc_b):
    n, din = g_flat.shape
    dout = fc_w.shape[1]
    feat = sum(dout // b for b in _BINS)
    dk = max(d for d in (4096, 2048, 1024, 512, 256, 128, din)
             if din % d == 0 and d <= din)
    return pl.pallas_call(
        _head_kernel,
        out_shape=jax.ShapeDtypeStruct((n, feat), jnp.float32),
        grid=(din // dk,),
        in_specs=[pl.BlockSpec((n, dk), lambda k: (0, k)),
                  pl.BlockSpec((dk, dout), lambda k: (k, 0)),
                  pl.BlockSpec((1, dout), lambda k: (0, 0))],
        out_specs=pl.BlockSpec((n, feat), lambda k: (0, 0)),
        scratch_shapes=[pltpu.VMEM((n, dout), jnp.float32)],
        compiler_params=pltpu.CompilerParams(
            dimension_semantics=("arbitrary",),
            vmem_limit_bytes=_VMEM),
    )(g_flat, fc_w, fc_b.reshape(1, dout))


def _gw(w, gout, S, sigma):
    _, _, cin, cout = w.shape
    main = jnp.zeros((3, S, cin, gout, cout), w.dtype)
    edge = jnp.zeros((3, 2, cin, gout, cout), w.dtype)
    for j in range(gout):
        for k in range(3):
            d = sigma * (j - 1 + k)
            if d < 0:
                edge = edge.at[:, 0, :, j, :].set(w[:, k, :, :])
            elif d >= S:
                edge = edge.at[:, 1, :, j, :].set(w[:, k, :, :])
            else:
                main = main.at[:, d, :, j, :].set(w[:, k, :, :])
    out = jnp.concatenate([main.reshape(3 * S * cin, gout * cout),
                           edge.reshape(3 * 2 * cin, gout * cout)], axis=0)
    return out.astype(jnp.bfloat16)


def _gw1(w, g):
    _, _, _, cout = w.shape
    out = jnp.zeros((5, g + 4, g, cout), w.dtype)
    for p in range(g):
        for k in range(5):
            out = out.at[:, p + k, p, :].set(w[:, k, 0, :])
    return out.reshape(5 * (g + 4), g * cout).astype(jnp.bfloat16)


def kernel(l1_w, l2_w, l3_w, l4_w, l5_w, l6_w, fc_w, fc_b, x):
    n, s, H, W = x.shape
    fb = max(d for d in range(1, 7) if s % d == 0)
    ws = [
        _gw1(l1_w, 8),
        _gw(l2_w, 8, 8, 1),
        _gw(l3_w, 4, 8, 2),
        _gw(l4_w, 4, 4, 1),
        _gw(l5_w, 2, 4, 2),
        _gw(l6_w, 2, 2, 1),
    ]
    g = _conv_stack(x, ws, fb=fb)
    feat = _head(g.reshape(n, -1), fc_w, fc_b)
    return feat[:, None, :], None


# speedup vs baseline: 1.3059x; 1.2175x over previous
import functools

import jax
import jax.numpy as jnp
from jax.experimental import pallas as pl
from jax.experimental.pallas import tpu as pltpu

_SLOPE = 0.01
_BINS = (1, 2, 4, 8, 16)
_VMEM = 64 * 1024 * 1024


def _lrelu(v):
    return jnp.where(v >= 0, v, _SLOPE * v)


def _gconv(x, w, *, cin, nout, left_slot, pool, out_dtype=jnp.bfloat16):
    F, h, G, L = x.shape
    cout = w.shape[1] // nout
    xp = jnp.pad(x, ((0, 0), (1, 1), (1, 1), (0, 0)))
    rows = [xp[:, kh:kh + h] for kh in range(3)]
    pieces = [r[:, :, 1:G + 1, :] for r in rows]
    for r in rows:
        pieces += [r[:, :, 0:G, left_slot * cin:(left_slot + 1) * cin],
                   r[:, :, 2:G + 2, 0:cin]]
    patch = jnp.concatenate(pieces, axis=-1)
    y = jnp.dot(patch.reshape(F * h * G, patch.shape[-1]), w,
                preferred_element_type=jnp.float32)
    y = _lrelu(y).astype(out_dtype).reshape(F, h, G, nout * cout)
    if pool:
        y = y.reshape(F, h // 2, 2, G, nout * cout)
        y = jnp.maximum(y[:, :, 0], y[:, :, 1])
        y = jnp.maximum(y, jnp.roll(y, -cout, axis=-1))
    return y


def _stack_kernel(x_ref, w1_ref, w2_ref, w3_ref, w4_ref, w5_ref, w6_ref,
                  o_ref, *, fb, H, W):
    G = W // 8
    xb = x_ref[0, 0].astype(jnp.bfloat16)
    patch = jnp.concatenate([xb[:, kh:kh + H] for kh in range(5)], axis=-1)
    y = jnp.dot(patch.reshape(fb * H, 5 * (W + 4)), w1_ref[...],
                preferred_element_type=jnp.float32)
    y = _lrelu(y).astype(jnp.bfloat16).reshape(fb, H, G, 8 * 32)

    y = _gconv(y, w2_ref[...], cin=32, nout=8, left_slot=7, pool=True)
    y = _gconv(y, w3_ref[...], cin=32, nout=4, left_slot=6, pool=False)
    y = _gconv(y, w4_ref[...], cin=64, nout=4, left_slot=3, pool=True)
    y = _gconv(y, w5_ref[...], cin=64, nout=2, left_slot=2, pool=False)
    y = _gconv(y, w6_ref[...], cin=128, nout=2, left_slot=1, pool=False,
               out_dtype=jnp.float32)

    m = jnp.max(y, axis=0)
    j = pl.program_id(1)

    @pl.when(j == 0)
    def _():
        o_ref[0] = m

    @pl.when(j > 0)
    def _():
        o_ref[0] = jnp.maximum(o_ref[0], m)


def _conv_stack(x, ws, *, fb):
    n, s, H, W = x.shape
    G = W // 8
    xp = jnp.pad(x, ((0, 0), (0, 0), (2, 2), (2, 2)))
    xp = xp.reshape(n, s // fb, fb, H + 4, W + 4)
    w_specs = [pl.BlockSpec(w.shape, lambda i, j, nd=w.ndim: (0,) * nd)
               for w in ws]
    kern = functools.partial(_stack_kernel, fb=fb, H=H, W=W)
    return pl.pallas_call(
        kern,
        out_shape=jax.ShapeDtypeStruct((n, H // 4, G, 256), jnp.float32),
        grid=(n, s // fb),
        in_specs=[pl.BlockSpec((1, 1, fb, H + 4, W + 4),
                               lambda i, j: (i, j, 0, 0, 0))] + w_specs,
        out_specs=pl.BlockSpec((1, H // 4, G, 256), lambda i, j: (i, 0, 0, 0)),
        compiler_params=pltpu.CompilerParams(
            dimension_semantics=("parallel", "arbitrary"),
            vmem_limit_bytes=_VMEM),
    )(xp, *ws)


def _head_kernel(g_ref, w_ref, b_ref, o_ref, acc_ref):
    k = pl.program_id(0)

    @pl.when(k == 0)
    def _():
        acc_ref[...] = jnp.zeros_like(acc_ref)

    acc_ref[...] += jnp.dot(g_ref[...].astype(jnp.bfloat16),
                            w_ref[...].astype(jnp.bfloat16),
                            preferred_element_type=jnp.float32)

    @pl.when(k == pl.num_programs(0) - 1)
    def _():
        z = acc_ref[...] + b_ref[...]
        dout = z.shape[1]
        off = 0
        for nb in _BINS:
            L = dout // nb
            ssum = z[:, :L]
            smax = z[:, :L]
            for b in range(1, nb):
                seg = z[:, b * L:(b + 1) * L]
                ssum = ssum + seg
                smax = jnp.maximum(smax, seg)
            o_ref[:, off:off + L] = ssum * (1.0 / nb) + smax
            off += L


def _head(g_flat, fc_w, fc_b):
    n, din = g_flat.shape
    dout = fc_w.shape[1]
    feat = sum(dout // b for b in _BINS)
    dk = max(d for d in (4096, 2048, 1024, 512, 256, 128, din)
             if din % d == 0 and d <= din)
    return pl.pallas_call(
        _head_kernel,
        out_shape=jax.ShapeDtypeStruct((n, feat), jnp.float32),
        grid=(din // dk,),
        in_specs=[pl.BlockSpec((n, dk), lambda k: (0, k)),
                  pl.BlockSpec((dk, dout), lambda k: (k, 0)),
                  pl.BlockSpec((1, dout), lambda k: (0, 0))],
        out_specs=pl.BlockSpec((n, feat), lambda k: (0, 0)),
        scratch_shapes=[pltpu.VMEM((n, dout), jnp.float32)],
        compiler_params=pltpu.CompilerParams(
            dimension_semantics=("arbitrary",),
            vmem_limit_bytes=_VMEM),
    )(g_flat, fc_w, fc_b.reshape(1, dout))


def _gw(w, gout, S, sigma):
    _, _, cin, cout = w.shape
    main = jnp.zeros((3, S, cin, gout, cout), w.dtype)
    edge = jnp.zeros((3, 2, cin, gout, cout), w.dtype)
    for j in range(gout):
        for k in range(3):
            d = sigma * (j - 1 + k)
            if d < 0:
                edge = edge.at[:, 0, :, j, :].set(w[:, k, :, :])
            elif d >= S:
                edge = edge.at[:, 1, :, j, :].set(w[:, k, :, :])
            else:
                main = main.at[:, d, :, j, :].set(w[:, k, :, :])
    out = jnp.concatenate([main.reshape(3 * S * cin, gout * cout),
                           edge.reshape(3 * 2 * cin, gout * cout)], axis=0)
    return out.astype(jnp.bfloat16)


def _gw1(w, W):
    _, _, _, cout = w.shape
    cols = jnp.arange(W + 4)[:, None]
    xs = jnp.arange(W)[None, :]
    kw = cols - xs
    band = jnp.take(w[:, :, 0, :], jnp.clip(kw, 0, 4), axis=1)
    band = jnp.where(((kw >= 0) & (kw <= 4))[None, :, :, None], band, 0.0)
    return band.reshape(5 * (W + 4), W * cout).astype(jnp.bfloat16)


def kernel(l1_w, l2_w, l3_w, l4_w, l5_w, l6_w, fc_w, fc_b, x):
    n, s, H, W = x.shape
    fb = max(d for d in range(1, 7) if s % d == 0)
    ws = [
        _gw1(l1_w, W),
        _gw(l2_w, 8, 8, 1),
        _gw(l3_w, 4, 8, 2),
        _gw(l4_w, 4, 4, 1),
        _gw(l5_w, 2, 4, 2),
        _gw(l6_w, 2, 2, 1),
    ]
    g = _conv_stack(x, ws, fb=fb)
    feat = _head(g.reshape(n, -1), fc_w, fc_b)
    return feat[:, None, :], None


# speedup vs baseline: 1.4671x; 1.1235x over previous
import functools

import jax
import jax.numpy as jnp
from jax.experimental import pallas as pl
from jax.experimental.pallas import tpu as pltpu

_SLOPE = 0.01
_BINS = (1, 2, 4, 8, 16)
_VMEM = 64 * 1024 * 1024


def _lrelu(v):
    return jnp.maximum(v, _SLOPE * v)


def _gconv(x, w, *, cin, nout, left_slot, pool):
    F, h, G, L = x.shape
    cout = w.shape[1] // nout
    xp = jnp.pad(x, ((0, 0), (1, 1), (1, 1), (0, 0)))
    rows = [xp[:, kh:kh + h] for kh in range(3)]
    pieces = [r[:, :, 1:G + 1, :] for r in rows]
    for r in rows:
        pieces += [r[:, :, 0:G, left_slot * cin:(left_slot + 1) * cin],
                   r[:, :, 2:G + 2, 0:cin]]
    patch = jnp.concatenate(pieces, axis=-1)
    patch = patch.reshape(F * h * G, patch.shape[-1]).astype(jnp.bfloat16)
    y = jnp.dot(patch, w, preferred_element_type=jnp.float32)
    y = _lrelu(y).reshape(F, h, G, nout * cout)
    if pool:
        y = y.reshape(F, h // 2, 2, G, nout * cout)
        y = jnp.maximum(y[:, :, 0], y[:, :, 1])
        y = jnp.maximum(y, jnp.roll(y, -cout, axis=-1))
    return y


def _stack_kernel(x_ref, w1_ref, w2_ref, w3_ref, w4_ref, w5_ref, w6_ref,
                  o_ref, *, fb, H, W):
    G = W // 8
    x = x_ref[0, 0]
    patch = jnp.concatenate([x[:, kh:kh + H] for kh in range(5)], axis=-1)
    patch = patch.reshape(fb * H, 5 * (W + 4)).astype(jnp.bfloat16)
    y = jnp.dot(patch, w1_ref[...], preferred_element_type=jnp.float32)
    y = _lrelu(y).reshape(fb, H, G, 8 * 32)

    y = _gconv(y, w2_ref[...], cin=32, nout=8, left_slot=7, pool=True)
    y = _gconv(y, w3_ref[...], cin=32, nout=4, left_slot=6, pool=False)
    y = _gconv(y, w4_ref[...], cin=64, nout=4, left_slot=3, pool=True)
    y = _gconv(y, w5_ref[...], cin=64, nout=2, left_slot=2, pool=False)
    y = _gconv(y, w6_ref[...], cin=128, nout=2, left_slot=1, pool=False)

    m = jnp.max(y, axis=0)
    j = pl.program_id(1)

    @pl.when(j == 0)
    def _():
        o_ref[0] = m

    @pl.when(j > 0)
    def _():
        o_ref[0] = jnp.maximum(o_ref[0], m)


def _conv_stack(x, ws, *, fb):
    n, s, H, W = x.shape
    G = W // 8
    xp = jnp.pad(x, ((0, 0), (0, 0), (2, 2), (2, 2)))
    xp = xp.reshape(n, s // fb, fb, H + 4, W + 4)
    w_specs = [pl.BlockSpec(w.shape, lambda i, j, nd=w.ndim: (0,) * nd)
               for w in ws]
    kern = functools.partial(_stack_kernel, fb=fb, H=H, W=W)
    return pl.pallas_call(
        kern,
        out_shape=jax.ShapeDtypeStruct((n, H // 4, G, 256), jnp.float32),
        grid=(n, s // fb),
        in_specs=[pl.BlockSpec((1, 1, fb, H + 4, W + 4),
                               lambda i, j: (i, j, 0, 0, 0))] + w_specs,
        out_specs=pl.BlockSpec((1, H // 4, G, 256), lambda i, j: (i, 0, 0, 0)),
        compiler_params=pltpu.CompilerParams(
            dimension_semantics=("parallel", "arbitrary"),
            vmem_limit_bytes=_VMEM),
    )(xp, *ws)


def _head_kernel(g_ref, w_ref, b_ref, o_ref, acc_ref):
    k = pl.program_id(0)

    @pl.when(k == 0)
    def _():
        acc_ref[...] = jnp.zeros_like(acc_ref)

    acc_ref[...] += jnp.dot(g_ref[...].astype(jnp.bfloat16),
                            w_ref[...].astype(jnp.bfloat16),
                            preferred_element_type=jnp.float32)

    @pl.when(k == pl.num_programs(0) - 1)
    def _():
        z = acc_ref[...] + b_ref[...]
        dout = z.shape[1]
        off = 0
        for nb in _BINS:
            L = dout // nb
            ssum = z[:, :L]
            smax = z[:, :L]
            for b in range(1, nb):
                seg = z[:, b * L:(b + 1) * L]
                ssum = ssum + seg
                smax = jnp.maximum(smax, seg)
            o_ref[:, off:off + L] = ssum * (1.0 / nb) + smax
            off += L


def _head(g_flat, fc_w, fc_b):
    n, din = g_flat.shape
    dout = fc_w.shape[1]
    feat = sum(dout // b for b in _BINS)
    dk = max(d for d in (4096, 2048, 1024, 512, 256, 128, din)
             if din % d == 0 and d <= din)
    return pl.pallas_call(
        _head_kernel,
        out_shape=jax.ShapeDtypeStruct((n, feat), jnp.float32),
        grid=(din // dk,),
        in_specs=[pl.BlockSpec((n, dk), lambda k: (0, k)),
                  pl.BlockSpec((dk, dout), lambda k: (k, 0)),
                  pl.BlockSpec((1, dout), lambda k: (0, 0))],
        out_specs=pl.BlockSpec((n, feat), lambda k: (0, 0)),
        scratch_shapes=[pltpu.VMEM((n, dout), jnp.float32)],
        compiler_params=pltpu.CompilerParams(
            dimension_semantics=("arbitrary",),
            vmem_limit_bytes=_VMEM),
    )(g_flat, fc_w, fc_b.reshape(1, dout))


def _gw(w, gout, S, sigma):
    _, _, cin, cout = w.shape
    main = jnp.zeros((3, S, cin, gout, cout), w.dtype)
    edge = jnp.zeros((3, 2, cin, gout, cout), w.dtype)
    for j in range(gout):
        for k in range(3):
            d = sigma * (j - 1 + k)
            if d < 0:
                edge = edge.at[:, 0, :, j, :].set(w[:, k, :, :])
            elif d >= S:
                edge = edge.at[:, 1, :, j, :].set(w[:, k, :, :])
            else:
                main = main.at[:, d, :, j, :].set(w[:, k, :, :])
    out = jnp.concatenate([main.reshape(3 * S * cin, gout * cout),
                           edge.reshape(3 * 2 * cin, gout * cout)], axis=0)
    return out.astype(jnp.bfloat16)


def _gw1(w, W):
    _, _, _, cout = w.shape
    cols = jnp.arange(W + 4)[:, None]
    xs = jnp.arange(W)[None, :]
    kw = cols - xs
    band = jnp.take(w[:, :, 0, :], jnp.clip(kw, 0, 4), axis=1)
    band = jnp.where(((kw >= 0) & (kw <= 4))[None, :, :, None], band, 0.0)
    return band.reshape(5 * (W + 4), W * cout).astype(jnp.bfloat16)


def kernel(l1_w, l2_w, l3_w, l4_w, l5_w, l6_w, fc_w, fc_b, x):
    n, s, H, W = x.shape
    fb = max(d for d in range(1, 7) if s % d == 0)
    ws = [
        _gw1(l1_w, W),
        _gw(l2_w, 8, 8, 1),
        _gw(l3_w, 4, 8, 2),
        _gw(l4_w, 4, 4, 1),
        _gw(l5_w, 2, 4, 2),
        _gw(l6_w, 2, 2, 1),
    ]
    g = _conv_stack(x, ws, fb=fb)
    feat = _head(g.reshape(n, -1), fc_w, fc_b)
    return feat[:, None, :], None


# speedup vs baseline: 1.5095x; 1.0289x over previous
import functools

import jax
import jax.numpy as jnp
from jax.experimental import pallas as pl
from jax.experimental.pallas import tpu as pltpu

_SLOPE = 0.01
_BINS = (1, 2, 4, 8, 16)
_VMEM = 64 * 1024 * 1024


def _lrelu(v):
    return jnp.maximum(v, _SLOPE * v)


def _gconv(x, w, *, cin, nout, left_slot, pool):
    F, h, G, L = x.shape
    cout = w.shape[1] // nout
    xp = jnp.pad(x, ((0, 0), (1, 1), (1, 1), (0, 0)))
    rows = [xp[:, kh:kh + h] for kh in range(3)]
    pieces = [r[:, :, 1:G + 1, :] for r in rows]
    for r in rows:
        pieces += [r[:, :, 0:G, left_slot * cin:(left_slot + 1) * cin],
                   r[:, :, 2:G + 2, 0:cin]]
    patch = jnp.concatenate(pieces, axis=-1)
    patch = patch.reshape(F * h * G, patch.shape[-1]).astype(jnp.bfloat16)
    y = jnp.dot(patch, w, preferred_element_type=jnp.float32)
    y = _lrelu(y).reshape(F, h, G, nout * cout)
    if pool:
        y = y.reshape(F, h // 2, 2, G, nout * cout)
        y = jnp.maximum(y[:, :, 0], y[:, :, 1])
        y = jnp.maximum(y, jnp.roll(y, -cout, axis=-1))
    return y


def _stack_kernel(x_ref, w1_ref, w2_ref, w3_ref, w4_ref, w5_ref, w6_ref,
                  o_ref, *, fb, H, W):
    G = W // 8
    x = x_ref[0, 0]
    patch = jnp.concatenate([x[:, kh:kh + H] for kh in range(5)], axis=-1)
    patch = patch.reshape(fb * H, 5 * (W + 4)).astype(jnp.bfloat16)
    y = jnp.dot(patch, w1_ref[...], preferred_element_type=jnp.float32)
    y = _lrelu(y).reshape(fb, H, G, 8 * 32)

    y = _gconv(y, w2_ref[...], cin=32, nout=8, left_slot=7, pool=True)
    y = _gconv(y, w3_ref[...], cin=32, nout=4, left_slot=6, pool=False)
    y = _gconv(y, w4_ref[...], cin=64, nout=4, left_slot=3, pool=True)
    y = _gconv(y, w5_ref[...], cin=64, nout=2, left_slot=2, pool=False)
    y = _gconv(y, w6_ref[...], cin=128, nout=2, left_slot=1, pool=False)

    m = jnp.max(y, axis=0)
    j = pl.program_id(1)

    @pl.when(j == 0)
    def _():
        o_ref[0] = m

    @pl.when(j > 0)
    def _():
        o_ref[0] = jnp.maximum(o_ref[0], m)


def _conv_stack(x, ws, *, fb):
    n, s, H, W = x.shape
    G = W // 8
    xp = jnp.pad(x, ((0, 0), (0, 0), (2, 2), (2, 2)))
    xp = xp.reshape(n, s // fb, fb, H + 4, W + 4)
    w_specs = [pl.BlockSpec(w.shape, lambda i, j, nd=w.ndim: (0,) * nd)
               for w in ws]
    kern = functools.partial(_stack_kernel, fb=fb, H=H, W=W)
    return pl.pallas_call(
        kern,
        out_shape=jax.ShapeDtypeStruct((n, H // 4, G, 256), jnp.float32),
        grid=(n, s // fb),
        in_specs=[pl.BlockSpec((1, 1, fb, H + 4, W + 4),
                               lambda i, j: (i, j, 0, 0, 0))] + w_specs,
        out_specs=pl.BlockSpec((1, H // 4, G, 256), lambda i, j: (i, 0, 0, 0)),
        compiler_params=pltpu.CompilerParams(
            dimension_semantics=("parallel", "arbitrary"),
            vmem_limit_bytes=_VMEM),
    )(xp, *ws)


def _head_kernel(g_ref, w_ref, b_ref, o_ref, acc_ref):
    k = pl.program_id(0)

    @pl.when(k == 0)
    def _():
        acc_ref[...] = jnp.zeros_like(acc_ref)

    acc_ref[...] += jnp.dot(g_ref[...].astype(jnp.bfloat16),
                            w_ref[...].astype(jnp.bfloat16),
                            preferred_element_type=jnp.float32)

    @pl.when(k == pl.num_programs(0) - 1)
    def _():
        z = acc_ref[...] + b_ref[...]
        dout = z.shape[1]
        off = 0
        for nb in _BINS:
            L = dout // nb
            ssum = z[:, :L]
            smax = z[:, :L]
            for b in range(1, nb):
                seg = z[:, b * L:(b + 1) * L]
                ssum = ssum + seg
                smax = jnp.maximum(smax, seg)
            o_ref[:, off:off + L] = ssum * (1.0 / nb) + smax
            off += L


def _head(g_flat, fc_w, fc_b):
    n, din = g_flat.shape
    dout = fc_w.shape[1]
    feat = sum(dout // b for b in _BINS)
    dk = max(d for d in (4096, 2048, 1024, 512, 256, 128, din)
             if din % d == 0 and d <= din)
    return pl.pallas_call(
        _head_kernel,
        out_shape=jax.ShapeDtypeStruct((n, feat), jnp.float32),
        grid=(din // dk,),
        in_specs=[pl.BlockSpec((n, dk), lambda k: (0, k)),
                  pl.BlockSpec((dk, dout), lambda k: (k, 0)),
                  pl.BlockSpec((1, dout), lambda k: (0, 0))],
        out_specs=pl.BlockSpec((n, feat), lambda k: (0, 0)),
        scratch_shapes=[pltpu.VMEM((n, dout), jnp.float32)],
        compiler_params=pltpu.CompilerParams(
            dimension_semantics=("arbitrary",),
            vmem_limit_bytes=_VMEM),
    )(g_flat, fc_w, fc_b.reshape(1, dout))


def _gw(w, gout, S, sigma):
    _, _, cin, cout = w.shape
    ds = jnp.arange(S)[:, None]
    js = jnp.arange(gout)[None, :]
    k = ds // sigma + 1 - js
    ok = (k >= 0) & (k <= 2) & (ds % sigma == 0)
    main = jnp.take(w, jnp.clip(k, 0, 2), axis=1)
    main = jnp.where(ok[None, :, :, None, None], main, 0.0)
    main = main.transpose(0, 1, 3, 2, 4)
    jhot = (js[0] == 0)[:, None]
    lhot = (js[0] == gout - 1)[:, None]
    left = w[:, 0, :, None, :] * jhot[None, None]
    right = w[:, 2, :, None, :] * lhot[None, None]
    edge = jnp.stack([left, right], axis=1)
    out = jnp.concatenate([main.reshape(3 * S * cin, gout * cout),
                           edge.reshape(3 * 2 * cin, gout * cout)], axis=0)
    return out.astype(jnp.bfloat16)


def _gw1(w, W):
    _, _, _, cout = w.shape
    cols = jnp.arange(W + 4)[:, None]
    xs = jnp.arange(W)[None, :]
    kw = cols - xs
    band = jnp.take(w[:, :, 0, :], jnp.clip(kw, 0, 4), axis=1)
    band = jnp.where(((kw >= 0) & (kw <= 4))[None, :, :, None], band, 0.0)
    return band.reshape(5 * (W + 4), W * cout).astype(jnp.bfloat16)


def kernel(l1_w, l2_w, l3_w, l4_w, l5_w, l6_w, fc_w, fc_b, x):
    n, s, H, W = x.shape
    fb = max(d for d in range(1, 7) if s % d == 0)
    ws = [
        _gw1(l1_w, W),
        _gw(l2_w, 8, 8, 1),
        _gw(l3_w, 4, 8, 2),
        _gw(l4_w, 4, 4, 1),
        _gw(l5_w, 2, 4, 2),
        _gw(l6_w, 2, 2, 1),
    ]
    g = _conv_stack(x, ws, fb=fb)
    feat = _head(g.reshape(n, -1), fc_w, fc_b)
    return feat[:, None, :], None
